# baseline (speedup 1.0000x reference)
import functools

import jax
import jax.numpy as jnp
from jax.experimental import pallas as pl
from jax.experimental.pallas import tpu as pltpu


def _scale_kernel(x_ref, w1_ref, b1_ref, w2_ref, b2x2_ref, scale_ref, *,
                  inv_hw):
    x = x_ref[...]
    bt = x.shape[0]

    avg = jnp.sum(x, axis=-1, dtype=jnp.float32) * inv_hw
    mx = jnp.max(x, axis=-1)

    pooled = jnp.concatenate([avg, mx], axis=0)
    h = jnp.dot(pooled, w1_ref[...], preferred_element_type=jnp.float32)
    h = jnp.maximum(h + b1_ref[...], 0.0)
    hs = h[:bt] + h[bt:]
    att = jnp.dot(hs, w2_ref[...], preferred_element_type=jnp.float32)
    scale_ref[...] = jax.nn.sigmoid(att + b2x2_ref[...])


def kernel(x, w1, b1, w2, b2):
    B, C, H, W = x.shape
    Ch = w1.shape[1]
    HW = H * W

    w1_f = w1.astype(jnp.float32)
    w2_f = w2.astype(jnp.float32)
    b1_2d = b1.reshape(1, Ch).astype(jnp.float32)
    b2x2 = (b2 * 2.0).reshape(1, C).astype(jnp.float32)

    x_flat = x.reshape(B, C, HW)

    bt = 8
    body = functools.partial(_scale_kernel, inv_hw=1.0 / HW)
    scale = pl.pallas_call(
        body,
        out_shape=jax.ShapeDtypeStruct((B, C), jnp.float32),
        grid=(B // bt,),
        in_specs=[
            pl.BlockSpec((bt, C, HW), lambda b: (b, 0, 0)),
            pl.BlockSpec((C, Ch), lambda b: (0, 0)),
            pl.BlockSpec((1, Ch), lambda b: (0, 0)),
            pl.BlockSpec((Ch, C), lambda b: (0, 0)),
            pl.BlockSpec((1, C), lambda b: (0, 0)),
        ],
        out_specs=pl.BlockSpec((bt, C), lambda b: (b, 0)),
        compiler_params=pltpu.CompilerParams(
            dimension_semantics=("parallel",),
            vmem_limit_bytes=int(48 * 1024 * 1024),
        ),
    )(x_flat, w1_f, b1_2d, w2_f, b2x2)

    return x * scale[:, :, None, None]
